# baseline (speedup 1.0000x reference)
import functools

import jax
import jax.numpy as jnp
from jax import lax
from jax.experimental import pallas as pl
from jax.experimental.pallas import tpu as pltpu

_BG = 4


def _mask_sample(xb, k):
    _, s, l = xb.shape
    hw = s * l

    e8 = jnp.sum(xb, axis=0)
    e_row = e8.reshape(1, hw)

    bits = lax.bitcast_convert_type(e_row, jnp.uint32)
    sign = bits >> 31
    key_row = bits ^ jnp.where(
        sign == 1, jnp.uint32(0xFFFFFFFF), jnp.uint32(0x80000000)
    )
    key_col = key_row.reshape(hw, 1)

    i_row = lax.broadcasted_iota(jnp.int32, (1, hw), 1)
    j_col = lax.broadcasted_iota(jnp.int32, (hw, 1), 0)
    beats = (key_col > key_row) | ((key_col == key_row) & (j_col < i_row))
    cnt = jnp.sum(beats.astype(jnp.int32), axis=0, keepdims=True)

    mask = jnp.where(cnt < jnp.int32(k), jnp.float32(1.0), jnp.float32(0.0))
    return xb * mask.reshape(s, l)[None]


def _fused_body(x_ref, o_ref, *, k):
    for i in range(x_ref.shape[0]):
        o_ref[i] = _mask_sample(x_ref[i], k)


def kernel(x):
    B, C, H, W = x.shape
    HW = H * W
    k = int(HW * 0.5)
    assert HW % 128 == 0 and B % _BG == 0
    S = HW // 128
    xr = x.reshape(B, C, S, 128)

    out = pl.pallas_call(
        functools.partial(_fused_body, k=k),
        grid=(B // _BG,),
        in_specs=[pl.BlockSpec((_BG, C, S, 128), lambda b: (b, 0, 0, 0))],
        out_specs=pl.BlockSpec((_BG, C, S, 128), lambda b: (b, 0, 0, 0)),
        out_shape=jax.ShapeDtypeStruct((B, C, S, 128), jnp.float32),
        compiler_params=pltpu.CompilerParams(
            dimension_semantics=("arbitrary",),
        ),
    )(xr)
    return out.reshape(B, C, H, W)
